# Initial kernel scaffold; baseline (speedup 1.0000x reference)
import functools

import jax
import jax.numpy as jnp
from jax import lax
from jax.experimental import pallas as pl
from jax.experimental.pallas import tpu as pltpu
from jax.experimental.pallas import tpu_sc as plsc

_NC = 2
_NS = 16
_LANES = 16


def _matmul(x, w):
    n, d_in = x.shape
    d_out = w.shape[1]
    bm = 1000

    def body(x_ref, w_ref, o_ref):
        o_ref[...] = jnp.dot(x_ref[...], w_ref[...],
                             preferred_element_type=jnp.float32)

    return pl.pallas_call(
        body,
        grid=(n // bm,),
        in_specs=[
            pl.BlockSpec((bm, d_in), lambda i: (i, 0)),
            pl.BlockSpec((d_in, d_out), lambda i: (0, 0)),
        ],
        out_specs=pl.BlockSpec((bm, d_out), lambda i: (i, 0)),
        out_shape=jax.ShapeDtypeStruct((n, d_out), jnp.float32),
    )(x, w)


def _sc_aggregate(prod, edge_index, edge_weight):
    n, d = prod.shape
    e_total = edge_index.shape[1]
    nw = _NC * _NS
    e_per_w = e_total // nw
    chunk = 80
    n_chunks = e_per_w // chunk
    assert e_per_w * nw == e_total and n_chunks * chunk == e_per_w
    rows_per_tile = n // _NS
    zr = 125
    assert rows_per_tile % zr == 0 and n % _NS == 0
    n_vregs = d // _LANES

    mesh = plsc.VectorSubcoreMesh(core_axis_name="c", subcore_axis_name="s")

    @functools.partial(
        pl.kernel,
        out_type=jax.ShapeDtypeStruct((_NC, n, d), jnp.float32),
        mesh=mesh,
        scratch_types=[
            pltpu.VMEM_SHARED((n, d), jnp.float32),
            pltpu.VMEM((chunk,), jnp.int32),
            pltpu.VMEM((chunk,), jnp.int32),
            pltpu.VMEM((chunk,), jnp.float32),
            pltpu.VMEM((chunk, d), jnp.float32),
            pltpu.VMEM((zr, d), jnp.float32),
            pltpu.SemaphoreType.DMA,
        ],
    )
    def agg(ei_hbm, ew_hbm, prod_hbm, out_hbm,
            acc, idx_s, idx_d, w_v, rows, zbuf, sem):
        c = lax.axis_index("c")
        s = lax.axis_index("s")
        wid = s * _NC + c

        zeros16 = jnp.zeros((_LANES,), jnp.float32)

        @pl.loop(0, zr)
        def _zero_fill(r):
            for j in range(n_vregs):
                zbuf[r, pl.ds(j * _LANES, _LANES)] = zeros16

        stripe = s * rows_per_tile

        @pl.loop(0, rows_per_tile // zr)
        def _zero_acc(k):
            pltpu.sync_copy(zbuf, acc.at[pl.ds(stripe + k * zr, zr), :])

        plsc.subcore_barrier()

        ebase = wid * e_per_w

        @pl.loop(0, n_chunks)
        def _chunk(g):
            base = ebase + g * chunk
            pltpu.sync_copy(ei_hbm.at[0, pl.ds(base, chunk)], idx_s)
            pltpu.sync_copy(ei_hbm.at[1, pl.ds(base, chunk)], idx_d)
            pltpu.sync_copy(ew_hbm.at[pl.ds(base, chunk)], w_v)
            pltpu.async_copy(prod_hbm.at[idx_s], rows, sem).wait()

            @pl.loop(0, chunk)
            def _scale(e):
                widx = jnp.full((_LANES,), e, jnp.int32)
                wv = plsc.load_gather(w_v, [widx])
                for j in range(n_vregs):
                    sl = pl.ds(j * _LANES, _LANES)
                    rows[e, sl] = rows[e, sl] * wv

            pltpu.sync_copy(rows, acc.at[idx_d], add=True)

        plsc.subcore_barrier()
        pltpu.sync_copy(acc.at[pl.ds(stripe, rows_per_tile), :],
                        out_hbm.at[c, pl.ds(stripe, rows_per_tile), :])

    return agg(edge_index, edge_weight, prod)


def _epilogue(partials, n_norm):
    _, n, d = partials.shape
    bm = 1000

    def body(p_ref, nn_ref, o_ref):
        o_ref[...] = jnp.maximum((p_ref[0] + p_ref[1]) * nn_ref[...], 0.0)

    return pl.pallas_call(
        body,
        grid=(n // bm,),
        in_specs=[
            pl.BlockSpec((2, bm, d), lambda i: (0, i, 0)),
            pl.BlockSpec((bm, 1), lambda i: (i, 0)),
        ],
        out_specs=pl.BlockSpec((bm, d), lambda i: (i, 0)),
        out_shape=jax.ShapeDtypeStruct((n, d), jnp.float32),
    )(partials, n_norm)


def kernel(x, edge_index, edge_weight, n_norm, W):
    prod = _matmul(x, W)
    partials = _sc_aggregate(prod, edge_index, edge_weight)
    return _epilogue(partials, n_norm)


# speedup vs baseline: 2.1068x; 2.1068x over previous
import functools

import jax
import jax.numpy as jnp
from jax import lax
from jax.experimental import pallas as pl
from jax.experimental.pallas import tpu as pltpu
from jax.experimental.pallas import tpu_sc as plsc

_NC = 2
_NS = 16
_LANES = 16


def _matmul(x, w):
    n, d_in = x.shape
    d_out = w.shape[1]
    dh = d_out // 2
    bm = 1000

    def body(x_ref, w_ref, o_ref):
        p = jnp.dot(x_ref[...], w_ref[...], preferred_element_type=jnp.float32)
        o_ref[0] = p[:, :dh]
        o_ref[1] = p[:, dh:]

    return pl.pallas_call(
        body,
        grid=(n // bm,),
        in_specs=[
            pl.BlockSpec((bm, d_in), lambda i: (i, 0)),
            pl.BlockSpec((d_in, d_out), lambda i: (0, 0)),
        ],
        out_specs=pl.BlockSpec((2, bm, dh), lambda i: (0, i, 0)),
        out_shape=jax.ShapeDtypeStruct((2, n, dh), jnp.float32),
    )(x, w)


def _sc_aggregate(prod2, edge_index_flat, edge_weight):
    n2, dh = prod2.shape
    n = n2 // 2
    e_total = edge_index_flat.shape[0] // 2
    e_per_tile = e_total // _NS
    chunk = 80
    n_chunks = e_per_tile // chunk
    assert e_per_tile * _NS == e_total and n_chunks * chunk == e_per_tile
    n_pad = -(-n // (8 * _NS)) * (8 * _NS)
    rows_per_tile = n_pad // _NS
    n_vregs = dh // _LANES

    mesh = plsc.VectorSubcoreMesh(core_axis_name="c", subcore_axis_name="s")

    @functools.partial(
        pl.kernel,
        out_type=jax.ShapeDtypeStruct((_NC, n_pad, dh), jnp.float32),
        mesh=mesh,
        compiler_params=pltpu.CompilerParams(use_tc_tiling_on_sc=False),
        scratch_types=[
            pltpu.VMEM_SHARED((n_pad, dh), jnp.float32),
            pltpu.VMEM((chunk,), jnp.int32),
            pltpu.VMEM((chunk,), jnp.int32),
            pltpu.VMEM((chunk,), jnp.float32),
            pltpu.VMEM((chunk, dh), jnp.float32),
            pltpu.VMEM((rows_per_tile, dh), jnp.float32),
            pltpu.SemaphoreType.DMA,
        ],
    )
    def agg(ei_hbm, ew_hbm, prod_hbm, out_hbm,
            acc, idx_s, idx_d, w_v, rows, zbuf, sem):
        c = lax.axis_index("c")
        s = lax.axis_index("s")

        zeros16 = jnp.zeros((_LANES,), jnp.float32)

        @pl.loop(0, rows_per_tile)
        def _zero_fill(r):
            for j in range(n_vregs):
                zbuf[r, pl.ds(j * _LANES, _LANES)] = zeros16

        stripe = s * rows_per_tile
        pltpu.sync_copy(zbuf, acc.at[pl.ds(stripe, rows_per_tile), :])

        plsc.subcore_barrier()

        ebase = s * e_per_tile
        tab_off = c * n

        @pl.loop(0, n_chunks)
        def _chunk(g):
            base = ebase + g * chunk
            pltpu.sync_copy(ei_hbm.at[pl.ds(base, chunk)], idx_s)
            pltpu.sync_copy(ei_hbm.at[pl.ds(e_total + base, chunk)], idx_d)
            pltpu.sync_copy(ew_hbm.at[pl.ds(base, chunk)], w_v)
            for k in range(chunk // _LANES):
                sl = pl.ds(k * _LANES, _LANES)
                idx_s[sl] = idx_s[sl] + tab_off
            pltpu.async_copy(prod_hbm.at[idx_s], rows, sem).wait()

            @pl.loop(0, chunk // _LANES)
            def _scale(k):
                w16 = w_v[pl.ds(k * _LANES, _LANES)]
                eb = k * _LANES
                dnums = lax.GatherDimensionNumbers(
                    offset_dims=(), collapsed_slice_dims=(0,),
                    start_index_map=(0,))
                for i in range(_LANES):
                    lane = jnp.full((_LANES, 1), i, jnp.int32)
                    wv = lax.gather(
                        w16, lane, dnums, slice_sizes=(1,),
                        mode=lax.GatherScatterMode.PROMISE_IN_BOUNDS)
                    for j in range(n_vregs):
                        sl = pl.ds(j * _LANES, _LANES)
                        rows[eb + i, sl] = rows[eb + i, sl] * wv

            pltpu.sync_copy(rows, acc.at[idx_d], add=True)

        plsc.subcore_barrier()
        pltpu.sync_copy(acc.at[pl.ds(stripe, rows_per_tile), :],
                        out_hbm.at[c, pl.ds(stripe, rows_per_tile), :])

    return agg(edge_index_flat, edge_weight, prod2)


def _epilogue(partials, n_norm):
    dh = partials.shape[2]
    n = n_norm.shape[0]
    bm = 1000

    def body(p_ref, nn_ref, o_ref):
        h = jnp.concatenate([p_ref[0], p_ref[1]], axis=1)
        o_ref[...] = jnp.maximum(h * nn_ref[...], 0.0)

    return pl.pallas_call(
        body,
        grid=(n // bm,),
        in_specs=[
            pl.BlockSpec((2, bm, dh), lambda i: (0, i, 0)),
            pl.BlockSpec((bm, 1), lambda i: (i, 0)),
        ],
        out_specs=pl.BlockSpec((bm, 2 * dh), lambda i: (i, 0)),
        out_shape=jax.ShapeDtypeStruct((n, 2 * dh), jnp.float32),
    )(partials, n_norm)


def kernel(x, edge_index, edge_weight, n_norm, W):
    prod = _matmul(x, W)
    prod2 = prod.reshape(-1, prod.shape[2])
    partials = _sc_aggregate(prod2, edge_index.reshape(-1), edge_weight)
    return _epilogue(partials, n_norm)


# speedup vs baseline: 9.2914x; 4.4101x over previous
import functools

import jax
import jax.numpy as jnp
from jax import lax
from jax.experimental import pallas as pl
from jax.experimental.pallas import tpu as pltpu
from jax.experimental.pallas import tpu_sc as plsc

_NC = 2
_NS = 16
_LANES = 16


def _matmul(x, w):
    n, d_in = x.shape
    d_out = w.shape[1]
    dh = d_out // 2
    bm = 1000

    def body(x_ref, w_ref, o_ref):
        p = jnp.dot(x_ref[...], w_ref[...], preferred_element_type=jnp.float32)
        o_ref[0] = p[:, :dh]
        o_ref[1] = p[:, dh:]

    return pl.pallas_call(
        body,
        grid=(n // bm,),
        in_specs=[
            pl.BlockSpec((bm, d_in), lambda i: (i, 0)),
            pl.BlockSpec((d_in, d_out), lambda i: (0, 0)),
        ],
        out_specs=pl.BlockSpec((2, bm, dh), lambda i: (0, i, 0)),
        out_shape=jax.ShapeDtypeStruct((2, n, dh), jnp.float32),
    )(x, w)


def _sc_aggregate(prod2, src3, dst3, ew3):
    n2, dh = prod2.shape
    n = n2 // 2
    _, n_chunks, chunk = src3.shape
    e_per_tile = n_chunks * chunk
    assert n_chunks % 2 == 0
    n_pad = -(-n // (8 * _NS)) * (8 * _NS)
    rows_per_tile = n_pad // _NS
    n_vregs = dh // _LANES
    zr = rows_per_tile // 4

    mesh = plsc.VectorSubcoreMesh(core_axis_name="c", subcore_axis_name="s")

    @functools.partial(
        pl.kernel,
        out_type=jax.ShapeDtypeStruct((_NC, n_pad, dh), jnp.float32),
        mesh=mesh,
        compiler_params=pltpu.CompilerParams(use_tc_tiling_on_sc=False),
        scratch_types=[
            pltpu.VMEM_SHARED((n_pad, dh), jnp.float32),
            pltpu.VMEM((n_chunks, chunk), jnp.int32),
            pltpu.VMEM((n_chunks, chunk), jnp.int32),
            pltpu.VMEM((n_chunks, chunk), jnp.float32),
            pltpu.VMEM((2, chunk, dh), jnp.float32),
            pltpu.VMEM((2, chunk, dh), jnp.float32),
            pltpu.VMEM((zr, dh), jnp.float32),
            pltpu.SemaphoreType.DMA,
            pltpu.SemaphoreType.DMA,
            pltpu.SemaphoreType.DMA,
            pltpu.SemaphoreType.DMA,
        ],
    )
    def agg(src_hbm, dst_hbm, ew_hbm, prod_hbm, out_hbm,
            acc, idx_s, idx_d, w_v, gbuf, sbuf, zbuf,
            sg0, sg1, ss0, ss1):
        c = lax.axis_index("c")
        s = lax.axis_index("s")
        sg = (sg0, sg1)
        ss = (ss0, ss1)

        pltpu.sync_copy(src_hbm.at[s], idx_s)
        pltpu.sync_copy(dst_hbm.at[s], idx_d)
        pltpu.sync_copy(ew_hbm.at[s], w_v)

        zeros16 = jnp.zeros((_LANES,), jnp.float32)

        @pl.loop(0, zr)
        def _zero_fill(r):
            for j in range(n_vregs):
                zbuf[r, pl.ds(j * _LANES, _LANES)] = zeros16

        stripe = s * rows_per_tile
        for q in range(4):
            pltpu.sync_copy(zbuf, acc.at[pl.ds(stripe + q * zr, zr), :])

        tab_off = c * n

        @pl.loop(0, n_chunks)
        def _off(r):
            for k in range(chunk // _LANES):
                sl = pl.ds(k * _LANES, _LANES)
                idx_s[r, sl] = idx_s[r, sl] + tab_off

        plsc.subcore_barrier()

        def start_gather(g, b):
            pltpu.async_copy(prod_hbm.at[idx_s.at[g]], gbuf.at[b], sg[b])

        def wait_gather(g, b):
            pltpu.make_async_copy(
                prod_hbm.at[idx_s.at[g]], gbuf.at[b], sg[b]).wait()

        def start_scatter(g, b):
            pltpu.async_copy(sbuf.at[b], acc.at[idx_d.at[g]], ss[b],
                             add=True)

        def wait_scatter(g, b):
            pltpu.make_async_copy(
                sbuf.at[b], acc.at[idx_d.at[g]], ss[b]).wait()

        start_gather(0, 0)
        start_gather(1, 1)

        dnums = lax.GatherDimensionNumbers(
            offset_dims=(), collapsed_slice_dims=(0,),
            start_index_map=(0,))

        @pl.loop(0, n_chunks, step=2)
        def _chunk(g0):
            for b in range(2):
                g = g0 + b
                wait_gather(g, b)

                @pl.when(g >= 2)
                def _():
                    wait_scatter(g - 2, b)

                @pl.loop(0, chunk // _LANES)
                def _scale(k):
                    w16 = w_v[g, pl.ds(k * _LANES, _LANES)]
                    eb = k * _LANES
                    for i in range(_LANES):
                        lane = jnp.full((_LANES, 1), i, jnp.int32)
                        wv = lax.gather(
                            w16, lane, dnums, slice_sizes=(1,),
                            mode=lax.GatherScatterMode.PROMISE_IN_BOUNDS)
                        for j in range(n_vregs):
                            sl = pl.ds(j * _LANES, _LANES)
                            sbuf[b, eb + i, sl] = gbuf[b, eb + i, sl] * wv

                @pl.when(g + 2 < n_chunks)
                def _():
                    start_gather(g + 2, b)

                start_scatter(g, b)

        for b in range(2):
            wait_scatter(n_chunks - 2 + b, b)

        plsc.subcore_barrier()
        pltpu.sync_copy(acc.at[pl.ds(stripe, rows_per_tile), :],
                        out_hbm.at[c, pl.ds(stripe, rows_per_tile), :])

    return agg(src3, dst3, ew3, prod2)


def _epilogue(partials, n_norm):
    dh = partials.shape[2]
    n = n_norm.shape[0]
    bm = 1000

    def body(p_ref, nn_ref, o_ref):
        h = jnp.concatenate([p_ref[0], p_ref[1]], axis=1)
        o_ref[...] = jnp.maximum(h * nn_ref[...], 0.0)

    return pl.pallas_call(
        body,
        grid=(n // bm,),
        in_specs=[
            pl.BlockSpec((2, bm, dh), lambda i: (0, i, 0)),
            pl.BlockSpec((bm, 1), lambda i: (i, 0)),
        ],
        out_specs=pl.BlockSpec((bm, 2 * dh), lambda i: (i, 0)),
        out_shape=jax.ShapeDtypeStruct((n, 2 * dh), jnp.float32),
    )(partials, n_norm)


def kernel(x, edge_index, edge_weight, n_norm, W):
    e_total = edge_index.shape[1]
    chunk = 80
    e_per_tile = e_total // _NS
    n_chunks = e_per_tile // chunk
    prod = _matmul(x, W)
    prod2 = prod.reshape(-1, prod.shape[2])
    src3 = edge_index[0].reshape(_NS, n_chunks, chunk)
    dst3 = edge_index[1].reshape(_NS, n_chunks, chunk)
    ew3 = edge_weight.reshape(_NS, n_chunks, chunk)
    partials = _sc_aggregate(prod2, src3, dst3, ew3)
    return _epilogue(partials, n_norm)


# speedup vs baseline: 9.2984x; 1.0008x over previous
import functools

import jax
import jax.numpy as jnp
from jax import lax
from jax.experimental import pallas as pl
from jax.experimental.pallas import tpu as pltpu
from jax.experimental.pallas import tpu_sc as plsc

_NC = 2
_NS = 16
_LANES = 16


def _matmul(x, w):
    n, d_in = x.shape
    d_out = w.shape[1]
    dh = d_out // 2
    bm = 1000

    def body(x_ref, w_ref, o_ref):
        p = jnp.dot(x_ref[...], w_ref[...], preferred_element_type=jnp.float32)
        o_ref[0] = p[:, :dh]
        o_ref[1] = p[:, dh:]

    return pl.pallas_call(
        body,
        grid=(n // bm,),
        in_specs=[
            pl.BlockSpec((bm, d_in), lambda i: (i, 0)),
            pl.BlockSpec((d_in, d_out), lambda i: (0, 0)),
        ],
        out_specs=pl.BlockSpec((2, bm, dh), lambda i: (0, i, 0)),
        out_shape=jax.ShapeDtypeStruct((2, n, dh), jnp.float32),
    )(x, w)


def _sc_aggregate(prod2, src3, dst3, ew3):
    n2, dh = prod2.shape
    n = n2 // 2
    _, n_chunks, chunk = src3.shape
    e_per_tile = n_chunks * chunk
    assert n_chunks % 2 == 0
    n_pad = -(-n // (8 * _NS)) * (8 * _NS)
    rows_per_tile = n_pad // _NS
    n_vregs = dh // _LANES
    zr = rows_per_tile // 4

    mesh = plsc.VectorSubcoreMesh(core_axis_name="c", subcore_axis_name="s")

    @functools.partial(
        pl.kernel,
        out_type=jax.ShapeDtypeStruct((_NC, n_pad, dh), jnp.float32),
        mesh=mesh,
        compiler_params=pltpu.CompilerParams(use_tc_tiling_on_sc=False),
        scratch_types=[
            pltpu.VMEM_SHARED((n_pad, dh), jnp.float32),
            pltpu.VMEM((n_chunks, chunk), jnp.int32),
            pltpu.VMEM((n_chunks, chunk), jnp.int32),
            pltpu.VMEM((n_chunks, chunk), jnp.float32),
            pltpu.VMEM((2, chunk, dh), jnp.float32),
            pltpu.VMEM((2, chunk, dh), jnp.float32),
            pltpu.VMEM((zr, dh), jnp.float32),
            pltpu.SemaphoreType.DMA,
            pltpu.SemaphoreType.DMA,
            pltpu.SemaphoreType.DMA,
            pltpu.SemaphoreType.DMA,
        ],
    )
    def agg(src_hbm, dst_hbm, ew_hbm, prod_hbm, out_hbm,
            acc, idx_s, idx_d, w_v, gbuf, sbuf, zbuf,
            sg0, sg1, ss0, ss1):
        c = lax.axis_index("c")
        s = lax.axis_index("s")
        sg = (sg0, sg1)
        ss = (ss0, ss1)

        pltpu.sync_copy(src_hbm.at[s], idx_s)
        pltpu.sync_copy(dst_hbm.at[s], idx_d)
        pltpu.sync_copy(ew_hbm.at[s], w_v)

        zeros16 = jnp.zeros((_LANES,), jnp.float32)

        @pl.loop(0, zr)
        def _zero_fill(r):
            for j in range(n_vregs):
                zbuf[r, pl.ds(j * _LANES, _LANES)] = zeros16

        stripe = s * rows_per_tile
        for q in range(4):
            pltpu.sync_copy(zbuf, acc.at[pl.ds(stripe + q * zr, zr), :])

        tab_off = c * n

        @pl.loop(0, n_chunks)
        def _off(r):
            for k in range(chunk // _LANES):
                sl = pl.ds(k * _LANES, _LANES)
                idx_s[r, sl] = idx_s[r, sl] + tab_off

        plsc.subcore_barrier()

        def start_gather(g, b):
            pltpu.async_copy(prod_hbm.at[idx_s.at[g]], gbuf.at[b], sg[b])

        def wait_gather(g, b):
            pltpu.make_async_copy(
                prod_hbm.at[idx_s.at[g]], gbuf.at[b], sg[b]).wait()

        def start_scatter(g, b):
            pltpu.async_copy(sbuf.at[b], acc.at[idx_d.at[g]], ss[b],
                             add=True)

        def wait_scatter(g, b):
            pltpu.make_async_copy(
                sbuf.at[b], acc.at[idx_d.at[g]], ss[b]).wait()

        start_gather(0, 0)
        start_gather(1, 1)

        dnums = lax.GatherDimensionNumbers(
            offset_dims=(), collapsed_slice_dims=(0,),
            start_index_map=(0,))

        @pl.loop(0, n_chunks, step=2)
        def _chunk(g0):
            for b in range(2):
                g = g0 + b
                wait_gather(g, b)

                @pl.when(g >= 2)
                def _():
                    wait_scatter(g - 2, b)

                for k in range(chunk // _LANES):
                    w16 = w_v[g, pl.ds(k * _LANES, _LANES)]
                    eb = k * _LANES
                    for i in range(_LANES):
                        lane = jnp.full((_LANES, 1), i, jnp.int32)
                        wv = lax.gather(
                            w16, lane, dnums, slice_sizes=(1,),
                            mode=lax.GatherScatterMode.PROMISE_IN_BOUNDS)
                        for j in range(n_vregs):
                            sl = pl.ds(j * _LANES, _LANES)
                            sbuf[b, eb + i, sl] = gbuf[b, eb + i, sl] * wv

                @pl.when(g + 2 < n_chunks)
                def _():
                    start_gather(g + 2, b)

                start_scatter(g, b)

        for b in range(2):
            wait_scatter(n_chunks - 2 + b, b)

        plsc.subcore_barrier()
        pltpu.sync_copy(acc.at[pl.ds(stripe, rows_per_tile), :],
                        out_hbm.at[c, pl.ds(stripe, rows_per_tile), :])

    return agg(src3, dst3, ew3, prod2)


def _epilogue(partials, n_norm):
    dh = partials.shape[2]
    n = n_norm.shape[0]
    bm = 1000

    def body(p_ref, nn_ref, o_ref):
        h = jnp.concatenate([p_ref[0], p_ref[1]], axis=1)
        o_ref[...] = jnp.maximum(h * nn_ref[...], 0.0)

    return pl.pallas_call(
        body,
        grid=(n // bm,),
        in_specs=[
            pl.BlockSpec((2, bm, dh), lambda i: (0, i, 0)),
            pl.BlockSpec((bm, 1), lambda i: (i, 0)),
        ],
        out_specs=pl.BlockSpec((bm, 2 * dh), lambda i: (i, 0)),
        out_shape=jax.ShapeDtypeStruct((n, 2 * dh), jnp.float32),
    )(partials, n_norm)


def kernel(x, edge_index, edge_weight, n_norm, W):
    e_total = edge_index.shape[1]
    chunk = 80
    e_per_tile = e_total // _NS
    n_chunks = e_per_tile // chunk
    prod = _matmul(x, W)
    prod2 = prod.reshape(-1, prod.shape[2])
    src3 = edge_index[0].reshape(_NS, n_chunks, chunk)
    dst3 = edge_index[1].reshape(_NS, n_chunks, chunk)
    ew3 = edge_weight.reshape(_NS, n_chunks, chunk)
    partials = _sc_aggregate(prod2, src3, dst3, ew3)
    return _epilogue(partials, n_norm)


# speedup vs baseline: 10.0436x; 1.0801x over previous
import functools

import jax
import jax.numpy as jnp
from jax import lax
from jax.experimental import pallas as pl
from jax.experimental.pallas import tpu as pltpu
from jax.experimental.pallas import tpu_sc as plsc

_NC = 2
_NS = 16
_LANES = 16


def _matmul(x, w):
    n, d_in = x.shape
    d_out = w.shape[1]
    dh = d_out // 2
    bm = 1000

    def body(x_ref, w_ref, o_ref):
        p = jnp.dot(x_ref[...], w_ref[...], preferred_element_type=jnp.float32)
        o_ref[0] = p[:, :dh]
        o_ref[1] = p[:, dh:]

    return pl.pallas_call(
        body,
        grid=(n // bm,),
        in_specs=[
            pl.BlockSpec((bm, d_in), lambda i: (i, 0)),
            pl.BlockSpec((d_in, d_out), lambda i: (0, 0)),
        ],
        out_specs=pl.BlockSpec((2, bm, dh), lambda i: (0, i, 0)),
        out_shape=jax.ShapeDtypeStruct((2, n, dh), jnp.float32),
    )(x, w)


def _sc_aggregate(prod2, src3, dst3, ew3):
    n2, dh = prod2.shape
    n = n2 // 2
    _, n_chunks, chunk = src3.shape
    e_per_tile = n_chunks * chunk
    assert n_chunks % 2 == 0
    n_pad = -(-n // (8 * _NS)) * (8 * _NS)
    rows_per_tile = n_pad // _NS
    n_vregs = dh // _LANES
    zr = rows_per_tile // 4

    mesh = plsc.VectorSubcoreMesh(core_axis_name="c", subcore_axis_name="s")

    @functools.partial(
        pl.kernel,
        out_type=jax.ShapeDtypeStruct((_NC, n_pad, dh), jnp.float32),
        mesh=mesh,
        compiler_params=pltpu.CompilerParams(use_tc_tiling_on_sc=False),
        scratch_types=[
            pltpu.VMEM_SHARED((n_pad, dh), jnp.float32),
            pltpu.VMEM((n_chunks, chunk), jnp.int32),
            pltpu.VMEM((n_chunks, chunk), jnp.int32),
            pltpu.VMEM((n_chunks, chunk), jnp.float32),
            pltpu.VMEM((2, chunk, dh), jnp.float32),
            pltpu.VMEM((2, chunk, dh), jnp.float32),
            pltpu.VMEM((zr, dh), jnp.float32),
            pltpu.SemaphoreType.DMA,
            pltpu.SemaphoreType.DMA,
            pltpu.SemaphoreType.DMA,
            pltpu.SemaphoreType.DMA,
        ],
    )
    def agg(src_hbm, dst_hbm, ew_hbm, prod_hbm, out_hbm,
            acc, idx_s, idx_d, w_v, gbuf, sbuf, zbuf,
            sg0, sg1, ss0, ss1):
        c = lax.axis_index("c")
        s = lax.axis_index("s")
        sg = (sg0, sg1)
        ss = (ss0, ss1)

        pltpu.sync_copy(src_hbm.at[s], idx_s)
        pltpu.sync_copy(dst_hbm.at[s], idx_d)
        pltpu.sync_copy(ew_hbm.at[s], w_v)

        zeros16 = jnp.zeros((_LANES,), jnp.float32)

        @pl.loop(0, zr)
        def _zero_fill(r):
            for j in range(n_vregs):
                zbuf[r, pl.ds(j * _LANES, _LANES)] = zeros16

        stripe = s * rows_per_tile
        for q in range(4):
            pltpu.sync_copy(zbuf, acc.at[pl.ds(stripe + q * zr, zr), :])

        @pl.loop(0, n_chunks)
        def _off(r):
            for k in range(chunk // _LANES):
                sl = pl.ds(k * _LANES, _LANES)
                idx_s[r, sl] = idx_s[r, sl] * 2 + c

        plsc.subcore_barrier()

        def start_gather(g, b):
            pltpu.async_copy(prod_hbm.at[idx_s.at[g]], gbuf.at[b], sg[b])

        def wait_gather(g, b):
            pltpu.make_async_copy(
                prod_hbm.at[idx_s.at[g]], gbuf.at[b], sg[b]).wait()

        def start_scatter(g, b):
            pltpu.async_copy(sbuf.at[b], acc.at[idx_d.at[g]], ss[b],
                             add=True)

        def wait_scatter(g, b):
            pltpu.make_async_copy(
                sbuf.at[b], acc.at[idx_d.at[g]], ss[b]).wait()

        start_gather(0, 0)
        start_gather(1, 1)

        dnums = lax.GatherDimensionNumbers(
            offset_dims=(), collapsed_slice_dims=(0,),
            start_index_map=(0,))

        @pl.loop(0, n_chunks, step=2)
        def _chunk(g0):
            for b in range(2):
                g = g0 + b
                wait_gather(g, b)

                @pl.when(g >= 2)
                def _():
                    wait_scatter(g - 2, b)

                for k in range(chunk // _LANES):
                    w16 = w_v[g, pl.ds(k * _LANES, _LANES)]
                    eb = k * _LANES
                    for i in range(_LANES):
                        lane = jnp.full((_LANES, 1), i, jnp.int32)
                        wv = lax.gather(
                            w16, lane, dnums, slice_sizes=(1,),
                            mode=lax.GatherScatterMode.PROMISE_IN_BOUNDS)
                        for j in range(n_vregs):
                            sl = pl.ds(j * _LANES, _LANES)
                            sbuf[b, eb + i, sl] = gbuf[b, eb + i, sl] * wv

                @pl.when(g + 2 < n_chunks)
                def _():
                    start_gather(g + 2, b)

                start_scatter(g, b)

        for b in range(2):
            wait_scatter(n_chunks - 2 + b, b)

        plsc.subcore_barrier()
        pltpu.sync_copy(acc.at[pl.ds(stripe, rows_per_tile), :],
                        out_hbm.at[c, pl.ds(stripe, rows_per_tile), :])

    return agg(src3, dst3, ew3, prod2)


def _epilogue(partials, n_norm, w):
    dh = partials.shape[2]
    n = n_norm.shape[0]
    d = 2 * dh
    bm = 1000

    def body(p_ref, nn_ref, w_ref, o_ref):
        h = jnp.concatenate([p_ref[0], p_ref[1]], axis=1)
        prod = jnp.dot(h, w_ref[...], preferred_element_type=jnp.float32,
                       precision=lax.Precision.HIGHEST)
        o_ref[...] = jnp.maximum(prod * nn_ref[...], 0.0)

    return pl.pallas_call(
        body,
        grid=(n // bm,),
        in_specs=[
            pl.BlockSpec((2, bm, dh), lambda i: (0, i, 0)),
            pl.BlockSpec((bm, 1), lambda i: (i, 0)),
            pl.BlockSpec((d, d), lambda i: (0, 0)),
        ],
        out_specs=pl.BlockSpec((bm, d), lambda i: (i, 0)),
        out_shape=jax.ShapeDtypeStruct((n, d), jnp.float32),
    )(partials, n_norm, w)


def kernel(x, edge_index, edge_weight, n_norm, W):
    e_total = edge_index.shape[1]
    chunk = 80
    e_per_tile = e_total // _NS
    n_chunks = e_per_tile // chunk
    x2 = x.reshape(-1, x.shape[1] // 2)
    src3 = edge_index[0].reshape(_NS, n_chunks, chunk)
    dst3 = edge_index[1].reshape(_NS, n_chunks, chunk)
    ew3 = edge_weight.reshape(_NS, n_chunks, chunk)
    partials = _sc_aggregate(x2, src3, dst3, ew3)
    return _epilogue(partials, n_norm, W)


# speedup vs baseline: 10.4910x; 1.0445x over previous
import functools

import jax
import jax.numpy as jnp
from jax import lax
from jax.experimental import pallas as pl
from jax.experimental.pallas import tpu as pltpu
from jax.experimental.pallas import tpu_sc as plsc

_NC = 2
_NS = 16
_LANES = 16


def _matmul(x, w):
    n, d_in = x.shape
    d_out = w.shape[1]
    dh = d_out // 2
    bm = 1000

    def body(x_ref, w_ref, o_ref):
        p = jnp.dot(x_ref[...], w_ref[...], preferred_element_type=jnp.float32)
        o_ref[0] = p[:, :dh]
        o_ref[1] = p[:, dh:]

    return pl.pallas_call(
        body,
        grid=(n // bm,),
        in_specs=[
            pl.BlockSpec((bm, d_in), lambda i: (i, 0)),
            pl.BlockSpec((d_in, d_out), lambda i: (0, 0)),
        ],
        out_specs=pl.BlockSpec((2, bm, dh), lambda i: (0, i, 0)),
        out_shape=jax.ShapeDtypeStruct((2, n, dh), jnp.float32),
    )(x, w)


def _sc_aggregate(prod2, src3, dst3, ew3):
    n2, dh = prod2.shape
    n = n2 // 2
    _, n_chunks, chunk = src3.shape
    e_per_tile = n_chunks * chunk
    assert n_chunks % 2 == 0
    n_pad = -(-n // (8 * _NS)) * (8 * _NS)
    rows_per_tile = n_pad // _NS
    n_vregs = dh // _LANES
    zr = rows_per_tile // 4

    mesh = plsc.VectorSubcoreMesh(core_axis_name="c", subcore_axis_name="s")

    @functools.partial(
        pl.kernel,
        out_type=jax.ShapeDtypeStruct((_NC, n_pad, dh), jnp.float32),
        mesh=mesh,
        compiler_params=pltpu.CompilerParams(
            use_tc_tiling_on_sc=False, needs_layout_passes=False),
        scratch_types=[
            pltpu.VMEM_SHARED((n_pad, dh), jnp.float32),
            pltpu.VMEM((n_chunks, chunk), jnp.int32),
            pltpu.VMEM((n_chunks, chunk), jnp.int32),
            pltpu.VMEM((n_chunks, chunk), jnp.float32),
            pltpu.VMEM((2, chunk, dh), jnp.bfloat16),
            pltpu.VMEM((2, chunk, dh), jnp.float32),
            pltpu.VMEM((zr, dh), jnp.float32),
            pltpu.SemaphoreType.DMA,
            pltpu.SemaphoreType.DMA,
            pltpu.SemaphoreType.DMA,
            pltpu.SemaphoreType.DMA,
        ],
    )
    def agg(src_hbm, dst_hbm, ew_hbm, prod_hbm, out_hbm,
            acc, idx_s, idx_d, w_v, gbuf, sbuf, zbuf,
            sg0, sg1, ss0, ss1):
        c = lax.axis_index("c")
        s = lax.axis_index("s")
        sg = (sg0, sg1)
        ss = (ss0, ss1)

        pltpu.sync_copy(src_hbm.at[s], idx_s)
        pltpu.sync_copy(dst_hbm.at[s], idx_d)
        pltpu.sync_copy(ew_hbm.at[s], w_v)

        zeros16 = jnp.zeros((_LANES,), jnp.float32)

        @pl.loop(0, zr)
        def _zero_fill(r):
            for j in range(n_vregs):
                zbuf[r, pl.ds(j * _LANES, _LANES)] = zeros16

        stripe = s * rows_per_tile
        for q in range(4):
            pltpu.sync_copy(zbuf, acc.at[pl.ds(stripe + q * zr, zr), :])

        @pl.loop(0, n_chunks)
        def _off(r):
            for k in range(chunk // _LANES):
                sl = pl.ds(k * _LANES, _LANES)
                idx_s[r, sl] = idx_s[r, sl] * 2 + c

        plsc.subcore_barrier()

        def start_gather(g, b):
            pltpu.async_copy(prod_hbm.at[idx_s.at[g]], gbuf.at[b], sg[b])

        def wait_gather(g, b):
            pltpu.make_async_copy(
                prod_hbm.at[idx_s.at[g]], gbuf.at[b], sg[b]).wait()

        def start_scatter(g, b):
            pltpu.async_copy(sbuf.at[b], acc.at[idx_d.at[g]], ss[b],
                             add=True)

        def wait_scatter(g, b):
            pltpu.make_async_copy(
                sbuf.at[b], acc.at[idx_d.at[g]], ss[b]).wait()

        start_gather(0, 0)
        start_gather(1, 1)

        dnums = lax.GatherDimensionNumbers(
            offset_dims=(), collapsed_slice_dims=(0,),
            start_index_map=(0,))

        @pl.loop(0, n_chunks, step=2)
        def _chunk(g0):
            for b in range(2):
                g = g0 + b
                wait_gather(g, b)

                @pl.when(g >= 2)
                def _():
                    wait_scatter(g - 2, b)

                for k in range(chunk // _LANES):
                    w16 = w_v[g, pl.ds(k * _LANES, _LANES)]
                    eb = k * _LANES
                    for i in range(_LANES):
                        lane = jnp.full((_LANES, 1), i, jnp.int32)
                        wv = lax.gather(
                            w16, lane, dnums, slice_sizes=(1,),
                            mode=lax.GatherScatterMode.PROMISE_IN_BOUNDS)
                        for h in range(n_vregs // 2):
                            v = gbuf[b, eb + i, pl.ds(h * 2 * _LANES,
                                                      2 * _LANES)]
                            lo, hi = plsc.unpack(
                                v, format=plsc.PackFormat.INTERLEAVED)
                            base = h * 2 * _LANES
                            sbuf[b, eb + i, pl.ds(base, _LANES)] = lo * wv
                            sbuf[b, eb + i, pl.ds(base + _LANES, _LANES)] = (
                                hi * wv)

                @pl.when(g + 2 < n_chunks)
                def _():
                    start_gather(g + 2, b)

                start_scatter(g, b)

        for b in range(2):
            wait_scatter(n_chunks - 2 + b, b)

        plsc.subcore_barrier()
        pltpu.sync_copy(acc.at[pl.ds(stripe, rows_per_tile), :],
                        out_hbm.at[c, pl.ds(stripe, rows_per_tile), :])

    return agg(src3, dst3, ew3, prod2)


def _epilogue(partials, n_norm, w):
    dh = partials.shape[2]
    n = n_norm.shape[0]
    d = 2 * dh
    bm = 1000

    def body(p_ref, nn_ref, w_ref, o_ref):
        h = jnp.concatenate([p_ref[0], p_ref[1]], axis=1)
        prod = jnp.dot(h, w_ref[...], preferred_element_type=jnp.float32,
                       precision=lax.Precision.HIGHEST)
        o_ref[...] = jnp.maximum(prod * nn_ref[...], 0.0)

    return pl.pallas_call(
        body,
        grid=(n // bm,),
        in_specs=[
            pl.BlockSpec((2, bm, dh), lambda i: (0, i, 0)),
            pl.BlockSpec((bm, 1), lambda i: (i, 0)),
            pl.BlockSpec((d, d), lambda i: (0, 0)),
        ],
        out_specs=pl.BlockSpec((bm, d), lambda i: (i, 0)),
        out_shape=jax.ShapeDtypeStruct((n, d), jnp.float32),
    )(partials, n_norm, w)


def kernel(x, edge_index, edge_weight, n_norm, W):
    e_total = edge_index.shape[1]
    chunk = 80
    e_per_tile = e_total // _NS
    n_chunks = e_per_tile // chunk
    dh = x.shape[1] // 2
    x2 = x.astype(jnp.bfloat16).reshape(-1, dh)
    src3 = edge_index[0].reshape(_NS, n_chunks, chunk)
    dst3 = edge_index[1].reshape(_NS, n_chunks, chunk)
    ew3 = edge_weight.reshape(_NS, n_chunks, chunk)
    partials = _sc_aggregate(x2, src3, dst3, ew3)
    perm = []
    for c in range(2):
        for h in range(dh // 32):
            for q in range(2):
                for p in range(_LANES):
                    perm.append(64 * c + 32 * h + 2 * p + q)
    Wp = W[jnp.array(perm, dtype=jnp.int32), :]
    return _epilogue(partials, n_norm, Wp)


# speedup vs baseline: 10.9876x; 1.0473x over previous
import functools

import jax
import jax.numpy as jnp
from jax import lax
from jax.experimental import pallas as pl
from jax.experimental.pallas import tpu as pltpu
from jax.experimental.pallas import tpu_sc as plsc

_NC = 2
_NS = 16
_LANES = 16


def _matmul(x, w):
    n, d_in = x.shape
    d_out = w.shape[1]
    dh = d_out // 2
    bm = 1000

    def body(x_ref, w_ref, o_ref):
        p = jnp.dot(x_ref[...], w_ref[...], preferred_element_type=jnp.float32)
        o_ref[0] = p[:, :dh]
        o_ref[1] = p[:, dh:]

    return pl.pallas_call(
        body,
        grid=(n // bm,),
        in_specs=[
            pl.BlockSpec((bm, d_in), lambda i: (i, 0)),
            pl.BlockSpec((d_in, d_out), lambda i: (0, 0)),
        ],
        out_specs=pl.BlockSpec((2, bm, dh), lambda i: (0, i, 0)),
        out_shape=jax.ShapeDtypeStruct((2, n, dh), jnp.float32),
    )(x, w)


def _sc_aggregate(prod2, src3, dst3, ew3):
    n2, dh = prod2.shape
    n = n2 // 2
    _, n_chunks, chunk = src3.shape
    e_per_tile = n_chunks * chunk
    assert n_chunks % 2 == 0
    n_pad = -(-n // (8 * _NS)) * (8 * _NS)
    rows_per_tile = n_pad // _NS
    n_vregs = dh // _LANES
    zr = rows_per_tile // 4

    mesh = plsc.VectorSubcoreMesh(core_axis_name="c", subcore_axis_name="s")

    @functools.partial(
        pl.kernel,
        out_type=jax.ShapeDtypeStruct((_NC, n_pad, dh), jnp.float32),
        mesh=mesh,
        compiler_params=pltpu.CompilerParams(
            use_tc_tiling_on_sc=False, needs_layout_passes=False),
        scratch_types=[
            pltpu.VMEM_SHARED((n_pad, dh), jnp.float32),
            pltpu.VMEM((n_chunks, chunk), jnp.int32),
            pltpu.VMEM((n_chunks, chunk), jnp.int32),
            pltpu.VMEM((n_chunks, chunk), jnp.float32),
            pltpu.VMEM((2, chunk, dh), jnp.bfloat16),
            pltpu.VMEM((2, chunk, dh), jnp.float32),
            pltpu.VMEM((zr, dh), jnp.float32),
            pltpu.SemaphoreType.DMA,
            pltpu.SemaphoreType.DMA,
            pltpu.SemaphoreType.DMA,
            pltpu.SemaphoreType.DMA,
        ],
    )
    def agg(src_hbm, dst_hbm, ew_hbm, prod_hbm, out_hbm,
            acc, idx_s, idx_d, w_v, gbuf, sbuf, zbuf,
            sg0, sg1, ss0, ss1):
        c = lax.axis_index("c")
        s = lax.axis_index("s")
        sg = (sg0, sg1)
        ss = (ss0, ss1)

        d_src = pltpu.async_copy(src_hbm.at[s], idx_s, sg0)
        d_dst = pltpu.async_copy(dst_hbm.at[s], idx_d, sg1)
        d_ew = pltpu.async_copy(ew_hbm.at[s], w_v, ss0)

        zeros16 = jnp.zeros((_LANES,), jnp.float32)

        @pl.loop(0, zr)
        def _zero_fill(r):
            for j in range(n_vregs):
                zbuf[r, pl.ds(j * _LANES, _LANES)] = zeros16

        stripe = s * rows_per_tile
        for q in range(4):
            pltpu.sync_copy(zbuf, acc.at[pl.ds(stripe + q * zr, zr), :])

        d_src.wait()

        @pl.loop(0, n_chunks)
        def _off(r):
            for k in range(chunk // _LANES):
                sl = pl.ds(k * _LANES, _LANES)
                idx_s[r, sl] = idx_s[r, sl] * 2 + c

        d_dst.wait()
        d_ew.wait()
        plsc.subcore_barrier()

        def start_gather(g, b):
            pltpu.async_copy(prod_hbm.at[idx_s.at[g]], gbuf.at[b], sg[b])

        def wait_gather(g, b):
            pltpu.make_async_copy(
                prod_hbm.at[idx_s.at[g]], gbuf.at[b], sg[b]).wait()

        def start_scatter(g, b):
            pltpu.async_copy(sbuf.at[b], acc.at[idx_d.at[g]], ss[b],
                             add=True)

        def wait_scatter(g, b):
            pltpu.make_async_copy(
                sbuf.at[b], acc.at[idx_d.at[g]], ss[b]).wait()

        start_gather(0, 0)
        start_gather(1, 1)

        dnums = lax.GatherDimensionNumbers(
            offset_dims=(), collapsed_slice_dims=(0,),
            start_index_map=(0,))

        @pl.loop(0, n_chunks, step=2)
        def _chunk(g0):
            for b in range(2):
                g = g0 + b
                wait_gather(g, b)

                @pl.when(g >= 2)
                def _():
                    wait_scatter(g - 2, b)

                for k in range(chunk // _LANES):
                    w16 = w_v[g, pl.ds(k * _LANES, _LANES)]
                    eb = k * _LANES
                    for i in range(_LANES):
                        lane = jnp.full((_LANES, 1), i, jnp.int32)
                        wv = lax.gather(
                            w16, lane, dnums, slice_sizes=(1,),
                            mode=lax.GatherScatterMode.PROMISE_IN_BOUNDS)
                        for h in range(n_vregs // 2):
                            v = gbuf[b, eb + i, pl.ds(h * 2 * _LANES,
                                                      2 * _LANES)]
                            lo, hi = plsc.unpack(
                                v, format=plsc.PackFormat.INTERLEAVED)
                            base = h * 2 * _LANES
                            sbuf[b, eb + i, pl.ds(base, _LANES)] = lo * wv
                            sbuf[b, eb + i, pl.ds(base + _LANES, _LANES)] = (
                                hi * wv)

                @pl.when(g + 2 < n_chunks)
                def _():
                    start_gather(g + 2, b)

                start_scatter(g, b)

        for b in range(2):
            wait_scatter(n_chunks - 2 + b, b)

        plsc.subcore_barrier()
        pltpu.sync_copy(acc.at[pl.ds(stripe, rows_per_tile), :],
                        out_hbm.at[c, pl.ds(stripe, rows_per_tile), :])

    return agg(src3, dst3, ew3, prod2)


def _epilogue(partials, n_norm, w):
    dh = partials.shape[2]
    n = n_norm.shape[0]
    d = 2 * dh
    bm = 2000

    def body(p_ref, nn_ref, w_ref, o_ref):
        h = jnp.concatenate([p_ref[0], p_ref[1]], axis=1)
        prod = jnp.dot(h, w_ref[...], preferred_element_type=jnp.float32)
        o_ref[...] = jnp.maximum(prod * nn_ref[...], 0.0)

    return pl.pallas_call(
        body,
        grid=(n // bm,),
        in_specs=[
            pl.BlockSpec((2, bm, dh), lambda i: (0, i, 0)),
            pl.BlockSpec((bm, 1), lambda i: (i, 0)),
            pl.BlockSpec((d, d), lambda i: (0, 0)),
        ],
        out_specs=pl.BlockSpec((bm, d), lambda i: (i, 0)),
        out_shape=jax.ShapeDtypeStruct((n, d), jnp.float32),
    )(partials, n_norm, w)


def kernel(x, edge_index, edge_weight, n_norm, W):
    e_total = edge_index.shape[1]
    chunk = 80
    e_per_tile = e_total // _NS
    n_chunks = e_per_tile // chunk
    dh = x.shape[1] // 2
    x2 = x.astype(jnp.bfloat16).reshape(-1, dh)
    src3 = edge_index[0].reshape(_NS, n_chunks, chunk)
    dst3 = edge_index[1].reshape(_NS, n_chunks, chunk)
    ew3 = edge_weight.reshape(_NS, n_chunks, chunk)
    partials = _sc_aggregate(x2, src3, dst3, ew3)
    perm = []
    for c in range(2):
        for h in range(dh // 32):
            for q in range(2):
                for p in range(_LANES):
                    perm.append(64 * c + 32 * h + 2 * p + q)
    Wp = W[jnp.array(perm, dtype=jnp.int32), :]
    return _epilogue(partials, n_norm, Wp)
